# baseline (speedup 1.0000x reference)
import functools

import jax
import jax.numpy as jnp
from jax import lax
from jax.experimental import pallas as pl
from jax.experimental.pallas import tpu as pltpu
from jax.experimental.pallas import tpu_sc as plsc

N = 10000
E = 320000
D_IN = 128
H1 = 400
H2 = 200
OUT = 4

NC = 2
NS = 16
CHUNK = 64
NBUF = 4
N_PAD = 10240
E_PAD = 327680
SPLIT_WIDE = (316, 4)
SPLIT_NARROW = (240, 80)
ROWS_PER_TILE = N_PAD // NS


def _sc_segsum(table, idx_both, width, split):
    mesh = plsc.VectorSubcoreMesh(core_axis_name="c", subcore_axis_name="s")
    CPW0, CPW1 = split

    @functools.partial(
        pl.kernel,
        mesh=mesh,
        compiler_params=pltpu.CompilerParams(use_tc_tiling_on_sc=False),
        out_type=jax.ShapeDtypeStruct((NC, N_PAD, width), jnp.float32),
        scratch_types=(
            [pltpu.VMEM((2, CHUNK), jnp.int32) for _ in range(NBUF)]
            + [pltpu.VMEM((CHUNK, width), jnp.float32) for _ in range(NBUF)]
            + [pltpu.VMEM_SHARED((N_PAD, width), jnp.float32)]
            + [pltpu.SemaphoreType.DMA] * (3 * NBUF)
        ),
    )
    def k(table_hbm, idx_hbm, out_hbm, *scr):
        ib = scr[:NBUF]
        rows = scr[NBUF:2 * NBUF]
        acc = scr[2 * NBUF]
        gsem = scr[2 * NBUF + 1:2 * NBUF + 1 + NBUF]
        ssem = scr[2 * NBUF + 1 + NBUF:2 * NBUF + 1 + 2 * NBUF]
        isem = scr[2 * NBUF + 1 + 2 * NBUF:]

        c = lax.axis_index("c")
        s = lax.axis_index("s")
        cpw = jnp.where(c == 0, CPW0, CPW1)
        base = jnp.where(c == 0, s * CPW0, NS * CPW0 + s * CPW1)
        nrounds = cpw // NBUF

        for k_ in range(NBUF):
            pltpu.async_copy(idx_hbm.at[base + k_], ib[k_], isem[k_])

        def _zr(r, carry):
            for kk in range(width // 16):
                rows[0][r, pl.ds(kk * 16, 16)] = jnp.zeros((16,), jnp.float32)
            return carry
        lax.fori_loop(0, CHUNK, _zr, 0)
        for j in range(ROWS_PER_TILE // CHUNK):
            pltpu.sync_copy(rows[0], acc.at[pl.ds(s * ROWS_PER_TILE + j * CHUNK, CHUNK)])

        for k_ in range(NBUF):
            pltpu.make_async_copy(idx_hbm.at[base + k_], ib[k_], isem[k_]).wait()
            pltpu.async_copy(table_hbm.at[ib[k_].at[0]], rows[k_], gsem[k_])
        plsc.subcore_barrier()

        def _round(r, carry):
            j = r * NBUF
            for k_ in range(NBUF):
                pltpu.make_async_copy(
                    table_hbm.at[ib[k_].at[0]], rows[k_], gsem[k_]).wait()
                pltpu.async_copy(rows[k_], acc.at[ib[k_].at[1]], ssem[k_], add=True)
            for k_ in range(NBUF):
                pltpu.make_async_copy(
                    rows[k_], acc.at[ib[k_].at[1]], ssem[k_]).wait()

                @pl.when(j + k_ + NBUF < cpw)
                def _():
                    pltpu.async_copy(idx_hbm.at[base + j + k_ + NBUF], ib[k_], isem[k_])
            for k_ in range(NBUF):
                @pl.when(j + k_ + NBUF < cpw)
                def _():
                    pltpu.make_async_copy(
                        idx_hbm.at[base + j + k_ + NBUF], ib[k_], isem[k_]).wait()
                    pltpu.async_copy(table_hbm.at[ib[k_].at[0]], rows[k_], gsem[k_])
            return carry

        lax.fori_loop(0, nrounds, _round, 0)
        plsc.subcore_barrier()

        pltpu.sync_copy(
            acc.at[pl.ds(s * ROWS_PER_TILE, ROWS_PER_TILE)],
            out_hbm.at[c, pl.ds(s * ROWS_PER_TILE, ROWS_PER_TILE)],
        )

    return k(table, idx_both)


_R = 1000
_G = N // _R


def _row_spec(width):
    return pl.BlockSpec((_R, width), lambda i: (i, 0))


def _part_spec(width):
    return pl.BlockSpec((NC, _R, width), lambda i: (0, i, 0))


def _full_spec(a, b):
    return pl.BlockSpec((a, b), lambda i: (0, 0))


def _tc1(p1, x, w1lT, b1, w1rT, w2aT, w2bT):
    def body(p1_ref, x_ref, wl_ref, b1_ref, wr_ref, w2a_ref, w2b_ref,
             h1_ref, y2a_ref, y2b_ref, inv_ref):
        agg = p1_ref[0] + p1_ref[1]
        cnt = agg[:, 128:129]
        inv = 1.0 / jnp.maximum(cnt, 1.0)
        mean = agg[:, :128] * inv
        h1 = jnp.maximum(
            jnp.dot(mean, wl_ref[...], preferred_element_type=jnp.float32)
            + b1_ref[...]
            + jnp.dot(x_ref[...], wr_ref[...], preferred_element_type=jnp.float32),
            0.0,
        )
        h1_ref[...] = h1
        y2a_ref[...] = jnp.dot(h1, w2a_ref[...], preferred_element_type=jnp.float32)
        y2b_ref[...] = jnp.dot(h1, w2b_ref[...], preferred_element_type=jnp.float32)
        inv_ref[...] = jnp.broadcast_to(inv, (_R, 8))

    return pl.pallas_call(
        body,
        grid=(_G,),
        in_specs=[
            _part_spec(144), _row_spec(128), _full_spec(128, H1),
            _full_spec(1, H1), _full_spec(128, H1),
            _full_spec(H1, 112), _full_spec(H1, 112),
        ],
        out_specs=[_row_spec(H1), _row_spec(112), _row_spec(112), _row_spec(8)],
        out_shape=[
            jax.ShapeDtypeStruct((N, H1), jnp.float32),
            jax.ShapeDtypeStruct((N, 112), jnp.float32),
            jax.ShapeDtypeStruct((N, 112), jnp.float32),
            jax.ShapeDtypeStruct((N, 8), jnp.float32),
        ],
    )(p1, x, w1lT, b1, w1rT, w2aT, w2bT)


def _tc2(p2a, p2b, h1, inv8, w2rT, b2, w3lp):
    def body(p2a_ref, p2b_ref, h1_ref, inv_ref, wr_ref, b2_ref, w3_ref,
             h2_ref, y3_ref):
        agg2 = jnp.concatenate(
            [p2a_ref[0] + p2a_ref[1], p2b_ref[0] + p2b_ref[1]], axis=1
        )[:, :H2]
        inv = inv_ref[:, 0:1]
        h2 = jnp.maximum(
            agg2 * inv + b2_ref[...]
            + jnp.dot(h1_ref[...], wr_ref[...], preferred_element_type=jnp.float32),
            0.0,
        )
        h2_ref[...] = h2
        y3_ref[...] = jnp.dot(h2, w3_ref[...], preferred_element_type=jnp.float32)

    return pl.pallas_call(
        body,
        grid=(_G,),
        in_specs=[
            _part_spec(112), _part_spec(112), _row_spec(H1), _row_spec(8),
            _full_spec(H1, H2), _full_spec(1, H2), _full_spec(H2, 16),
        ],
        out_specs=[_row_spec(H2), _row_spec(16)],
        out_shape=[
            jax.ShapeDtypeStruct((N, H2), jnp.float32),
            jax.ShapeDtypeStruct((N, 16), jnp.float32),
        ],
    )(p2a, p2b, h1, inv8, w2rT, b2, w3lp)


def _tc3(p3, h2, inv8, w3rT, b3):
    def body(p3_ref, h2_ref, inv_ref, wr_ref, b3_ref, out_ref):
        agg3 = (p3_ref[0] + p3_ref[1])[:, :OUT]
        inv = inv_ref[:, 0:1]
        out_ref[...] = jnp.maximum(
            agg3 * inv + b3_ref[...]
            + jnp.dot(h2_ref[...], wr_ref[...], preferred_element_type=jnp.float32),
            0.0,
        )

    return pl.pallas_call(
        body,
        grid=(_G,),
        in_specs=[
            _part_spec(16), _row_spec(H2), _row_spec(8),
            _full_spec(H2, OUT), _full_spec(1, OUT),
        ],
        out_specs=_row_spec(OUT),
        out_shape=jax.ShapeDtypeStruct((N, OUT), jnp.float32),
    )(p3, h2, inv8, w3rT, b3)


def kernel(x, edge_index, W1l, b1, W1r, W2l, b2, W2r, W3l, b3, W3r):
    src = edge_index[0]
    dst = edge_index[1]
    pad = E_PAD - E
    src_p = jnp.concatenate([src, jnp.zeros((pad,), jnp.int32)]).reshape(-1, 1, CHUNK)
    dst_p = jnp.concatenate([dst, jnp.full((pad,), N, jnp.int32)]).reshape(-1, 1, CHUNK)
    idx_both = jnp.concatenate([src_p, dst_p], axis=1)

    xp = jnp.concatenate(
        [x, jnp.ones((N, 1), jnp.float32), jnp.zeros((N, 15), jnp.float32)], axis=1
    )

    w1lT = W1l.T
    w1rT = W1r.T
    w2lT = W2l.T
    w2aT = w2lT[:, :112]
    w2bT = jnp.pad(w2lT[:, 112:], ((0, 0), (0, 24)))
    w2rT = W2r.T
    w3lp = jnp.pad(W3l.T, ((0, 0), (0, 12)))
    w3rT = W3r.T
    b1r = b1.reshape(1, H1)
    b2r = b2.reshape(1, H2)
    b3r = b3.reshape(1, OUT)

    p1 = _sc_segsum(xp, idx_both, 144, SPLIT_WIDE)
    h1, y2a, y2b, inv8 = _tc1(p1, x, w1lT, b1r, w1rT, w2aT, w2bT)

    p2a = _sc_segsum(y2a, idx_both, 112, SPLIT_WIDE)
    p2b = _sc_segsum(y2b, idx_both, 112, SPLIT_WIDE)
    h2, y3 = _tc2(p2a, p2b, h1, inv8, w2rT, b2r, w3lp)

    p3 = _sc_segsum(y3, idx_both, 16, SPLIT_NARROW)
    return _tc3(p3, h2, inv8, w3rT, b3r)


# speedup vs baseline: 1.1597x; 1.1597x over previous
import functools

import jax
import jax.numpy as jnp
from jax import lax
from jax.experimental import pallas as pl
from jax.experimental.pallas import tpu as pltpu
from jax.experimental.pallas import tpu_sc as plsc

N = 10000
E = 320000
D_IN = 128
H1 = 400
H2 = 200
OUT = 4

NC = 2
NS = 16
CHUNK = 64
NBUF = 4
N_PAD = 10240
E_PAD = 327680
SPLIT_144 = (224, 96)
SPLIT_112 = (228, 92)
SPLIT_96 = (228, 92)
SPLIT_16 = (232, 88)
ROWS_PER_TILE = N_PAD // NS


def _sc_segsum(table, idx_both, width, split):
    mesh = plsc.VectorSubcoreMesh(core_axis_name="c", subcore_axis_name="s")
    CPW0, CPW1 = split

    @functools.partial(
        pl.kernel,
        mesh=mesh,
        compiler_params=pltpu.CompilerParams(use_tc_tiling_on_sc=False),
        out_type=jax.ShapeDtypeStruct((NC, N_PAD, width), jnp.float32),
        scratch_types=(
            [pltpu.VMEM((2, CHUNK), jnp.int32) for _ in range(NBUF)]
            + [pltpu.VMEM((CHUNK, width), jnp.float32) for _ in range(NBUF)]
            + [pltpu.VMEM_SHARED((N_PAD, width), jnp.float32)]
            + [pltpu.SemaphoreType.DMA] * (3 * NBUF)
        ),
    )
    def k(table_hbm, idx_hbm, out_hbm, *scr):
        ib = scr[:NBUF]
        rows = scr[NBUF:2 * NBUF]
        acc = scr[2 * NBUF]
        gsem = scr[2 * NBUF + 1:2 * NBUF + 1 + NBUF]
        ssem = scr[2 * NBUF + 1 + NBUF:2 * NBUF + 1 + 2 * NBUF]
        isem = scr[2 * NBUF + 1 + 2 * NBUF:]

        c = lax.axis_index("c")
        s = lax.axis_index("s")
        cpw = jnp.where(c == 0, CPW0, CPW1)
        base = jnp.where(c == 0, s * CPW0, NS * CPW0 + s * CPW1)
        nrounds = cpw // NBUF

        for k_ in range(NBUF):
            pltpu.async_copy(idx_hbm.at[base + k_], ib[k_], isem[k_])

        def _zr(r, carry):
            for kk in range(width // 16):
                rows[0][r, pl.ds(kk * 16, 16)] = jnp.zeros((16,), jnp.float32)
            return carry
        lax.fori_loop(0, CHUNK, _zr, 0)
        for j in range(ROWS_PER_TILE // CHUNK):
            pltpu.sync_copy(rows[0], acc.at[pl.ds(s * ROWS_PER_TILE + j * CHUNK, CHUNK)])

        for k_ in range(NBUF):
            pltpu.make_async_copy(idx_hbm.at[base + k_], ib[k_], isem[k_]).wait()
            pltpu.async_copy(table_hbm.at[ib[k_].at[0]], rows[k_], gsem[k_])
        plsc.subcore_barrier()

        def _round(r, carry):
            j = r * NBUF
            for k_ in range(NBUF):
                pltpu.make_async_copy(
                    table_hbm.at[ib[k_].at[0]], rows[k_], gsem[k_]).wait()
                pltpu.async_copy(rows[k_], acc.at[ib[k_].at[1]], ssem[k_], add=True)
            for k_ in range(NBUF):
                pltpu.make_async_copy(
                    rows[k_], acc.at[ib[k_].at[1]], ssem[k_]).wait()

                @pl.when(j + k_ + NBUF < cpw)
                def _():
                    pltpu.async_copy(idx_hbm.at[base + j + k_ + NBUF], ib[k_], isem[k_])
            for k_ in range(NBUF):
                @pl.when(j + k_ + NBUF < cpw)
                def _():
                    pltpu.make_async_copy(
                        idx_hbm.at[base + j + k_ + NBUF], ib[k_], isem[k_]).wait()
                    pltpu.async_copy(table_hbm.at[ib[k_].at[0]], rows[k_], gsem[k_])
            return carry

        lax.fori_loop(0, nrounds, _round, 0)
        plsc.subcore_barrier()

        pltpu.sync_copy(
            acc.at[pl.ds(s * ROWS_PER_TILE, ROWS_PER_TILE)],
            out_hbm.at[c, pl.ds(s * ROWS_PER_TILE, ROWS_PER_TILE)],
        )

    return k(table, idx_both)


_R = 1000
_G = N // _R


def _row_spec(width):
    return pl.BlockSpec((_R, width), lambda i: (i, 0))


def _part_spec(width):
    return pl.BlockSpec((NC, _R, width), lambda i: (0, i, 0))


def _full_spec(a, b):
    return pl.BlockSpec((a, b), lambda i: (0, 0))


def _tc1(p1, x, w1lT, b1, w1rT, w2aT, w2bT):
    def body(p1_ref, x_ref, wl_ref, b1_ref, wr_ref, w2a_ref, w2b_ref,
             h1_ref, y2a_ref, y2b_ref, inv_ref):
        agg = p1_ref[0] + p1_ref[1]
        cnt = agg[:, 128:129]
        inv = 1.0 / jnp.maximum(cnt, 1.0)
        mean = agg[:, :128] * inv
        h1 = jnp.maximum(
            jnp.dot(mean, wl_ref[...], preferred_element_type=jnp.float32)
            + b1_ref[...]
            + jnp.dot(x_ref[...], wr_ref[...], preferred_element_type=jnp.float32),
            0.0,
        )
        h1_ref[...] = h1
        y2a_ref[...] = jnp.dot(h1, w2a_ref[...], preferred_element_type=jnp.float32)
        y2b_ref[...] = jnp.dot(h1, w2b_ref[...], preferred_element_type=jnp.float32)
        inv_ref[...] = jnp.broadcast_to(inv, (_R, 8))

    return pl.pallas_call(
        body,
        grid=(_G,),
        in_specs=[
            _part_spec(144), _row_spec(128), _full_spec(128, H1),
            _full_spec(1, H1), _full_spec(128, H1),
            _full_spec(H1, 112), _full_spec(H1, 96),
        ],
        out_specs=[_row_spec(H1), _row_spec(112), _row_spec(96), _row_spec(8)],
        out_shape=[
            jax.ShapeDtypeStruct((N, H1), jnp.float32),
            jax.ShapeDtypeStruct((N, 112), jnp.float32),
            jax.ShapeDtypeStruct((N, 96), jnp.float32),
            jax.ShapeDtypeStruct((N, 8), jnp.float32),
        ],
    )(p1, x, w1lT, b1, w1rT, w2aT, w2bT)


def _tc2(p2a, p2b, h1, inv8, w2rT, b2, w3lp):
    def body(p2a_ref, p2b_ref, h1_ref, inv_ref, wr_ref, b2_ref, w3_ref,
             h2_ref, y3_ref):
        agg2 = jnp.concatenate(
            [p2a_ref[0] + p2a_ref[1], p2b_ref[0] + p2b_ref[1]], axis=1
        )[:, :H2]
        inv = inv_ref[:, 0:1]
        h2 = jnp.maximum(
            agg2 * inv + b2_ref[...]
            + jnp.dot(h1_ref[...], wr_ref[...], preferred_element_type=jnp.float32),
            0.0,
        )
        h2_ref[...] = h2
        y3_ref[...] = jnp.dot(h2, w3_ref[...], preferred_element_type=jnp.float32)

    return pl.pallas_call(
        body,
        grid=(_G,),
        in_specs=[
            _part_spec(112), _part_spec(96), _row_spec(H1), _row_spec(8),
            _full_spec(H1, H2), _full_spec(1, H2), _full_spec(H2, 16),
        ],
        out_specs=[_row_spec(H2), _row_spec(16)],
        out_shape=[
            jax.ShapeDtypeStruct((N, H2), jnp.float32),
            jax.ShapeDtypeStruct((N, 16), jnp.float32),
        ],
    )(p2a, p2b, h1, inv8, w2rT, b2, w3lp)


def _tc3(p3, h2, inv8, w3rT, b3):
    def body(p3_ref, h2_ref, inv_ref, wr_ref, b3_ref, out_ref):
        agg3 = (p3_ref[0] + p3_ref[1])[:, :OUT]
        inv = inv_ref[:, 0:1]
        out_ref[...] = jnp.maximum(
            agg3 * inv + b3_ref[...]
            + jnp.dot(h2_ref[...], wr_ref[...], preferred_element_type=jnp.float32),
            0.0,
        )

    return pl.pallas_call(
        body,
        grid=(_G,),
        in_specs=[
            _part_spec(16), _row_spec(H2), _row_spec(8),
            _full_spec(H2, OUT), _full_spec(1, OUT),
        ],
        out_specs=_row_spec(OUT),
        out_shape=jax.ShapeDtypeStruct((N, OUT), jnp.float32),
    )(p3, h2, inv8, w3rT, b3)


def kernel(x, edge_index, W1l, b1, W1r, W2l, b2, W2r, W3l, b3, W3r):
    src = edge_index[0]
    dst = edge_index[1]
    pad = E_PAD - E
    src_p = jnp.concatenate([src, jnp.zeros((pad,), jnp.int32)]).reshape(-1, 1, CHUNK)
    dst_p = jnp.concatenate([dst, jnp.full((pad,), N, jnp.int32)]).reshape(-1, 1, CHUNK)
    idx_both = jnp.concatenate([src_p, dst_p], axis=1)

    xp = jnp.concatenate(
        [x, jnp.ones((N, 1), jnp.float32), jnp.zeros((N, 15), jnp.float32)], axis=1
    )

    w1lT = W1l.T
    w1rT = W1r.T
    w2lT = W2l.T
    w2aT = w2lT[:, :112]
    w2bT = jnp.pad(w2lT[:, 112:], ((0, 0), (0, 8)))
    w2rT = W2r.T
    w3lp = jnp.pad(W3l.T, ((0, 0), (0, 12)))
    w3rT = W3r.T
    b1r = b1.reshape(1, H1)
    b2r = b2.reshape(1, H2)
    b3r = b3.reshape(1, OUT)

    p1 = _sc_segsum(xp, idx_both, 144, SPLIT_144)
    h1, y2a, y2b, inv8 = _tc1(p1, x, w1lT, b1r, w1rT, w2aT, w2bT)

    p2a = _sc_segsum(y2a, idx_both, 112, SPLIT_112)
    p2b = _sc_segsum(y2b, idx_both, 96, SPLIT_96)
    h2, y3 = _tc2(p2a, p2b, h1, inv8, w2rT, b2r, w3lp)

    p3 = _sc_segsum(y3, idx_both, 16, SPLIT_16)
    return _tc3(p3, h2, inv8, w3rT, b3r)
